# baseline (speedup 1.0000x reference)
import jax
import jax.numpy as jnp
from jax.experimental import pallas as pl

B, J, D = 1024, 26, 1000
BR = 64


def _body(x_ref, o_ref):
    xv = x_ref[...]
    iota = jax.lax.broadcasted_iota(jnp.int32, (BR, J, D), 2)
    o_ref[...] = (iota == xv[:, :, None]).astype(jnp.float32)


def kernel(x):
    out = pl.pallas_call(
        _body,
        grid=(B // BR,),
        in_specs=[pl.BlockSpec((BR, J), lambda i: (i, 0))],
        out_specs=pl.BlockSpec((BR, J, D), lambda i: (i, 0, 0)),
        out_shape=jax.ShapeDtypeStruct((B, J, D), jnp.float32),
    )(x)
    return out.reshape(B, J * D)


# speedup vs baseline: 1.0494x; 1.0494x over previous
import functools
import jax
import jax.numpy as jnp
from jax import lax
from jax.experimental import pallas as pl
from jax.experimental.pallas import tpu as pltpu, tpu_sc as plsc

B, J, D = 1024, 26, 1000
ROW = J * D
NBUF = 2

_info = plsc.get_sparse_core_info()
NC, NS, L = _info.num_cores, _info.num_subcores, _info.num_lanes
NW = NC * NS
RPW = B // NW


def _sc_body(x_hbm, out_hbm, x_v, buf0, buf1, sems):
    wid = lax.axis_index("s") * NC + lax.axis_index("c")
    base = wid * RPW
    bufs = [buf0, buf1]

    pltpu.sync_copy(x_hbm.at[pl.ds(base, RPW)], x_v)

    def _zero(i, _):
        z = jnp.zeros((L,), jnp.float32)
        buf0[pl.ds(i * L, L)] = z
        buf1[pl.ds(i * L, L)] = z
        return 0

    lax.fori_loop(0, ROW // L, _zero, 0)

    ones = jnp.ones((L,), jnp.float32)
    zeros = jnp.zeros((L,), jnp.float32)
    lane = lax.broadcasted_iota(jnp.int32, (L,), 0)
    off0 = lane * D
    off1 = (lane + (J - L)) * D

    def _flat_idx(r):
        v0 = x_v[r, pl.ds(0, L)]
        v1 = x_v[r, pl.ds(J - L, L)]
        return off0 + v0, off1 + v1

    copies = [None] * NBUF
    for r in range(RPW):
        b = r % NBUF
        buf = bufs[b]
        if copies[b] is not None:
            copies[b].wait()
            p0, p1 = _flat_idx(r - NBUF)
            plsc.store_scatter(buf, [p0], zeros)
            plsc.store_scatter(buf, [p1], zeros)
        i0, i1 = _flat_idx(r)
        plsc.store_scatter(buf, [i0], ones)
        plsc.store_scatter(buf, [i1], ones)
        cp = pltpu.make_async_copy(buf, out_hbm.at[base + r], sems.at[b])
        cp.start()
        copies[b] = cp
    for b in range(NBUF):
        copies[b].wait()


def kernel(x):
    mesh = plsc.VectorSubcoreMesh(core_axis_name="c", subcore_axis_name="s")
    f = pl.kernel(
        _sc_body,
        out_type=jax.ShapeDtypeStruct((B, ROW), jnp.float32),
        mesh=mesh,
        scratch_types=[
            pltpu.VMEM((RPW, J), jnp.int32),
            pltpu.VMEM((ROW,), jnp.float32),
            pltpu.VMEM((ROW,), jnp.float32),
            pltpu.SemaphoreType.DMA((NBUF,)),
        ],
        compiler_params=pltpu.CompilerParams(needs_layout_passes=False),
    )
    return f(x)


# speedup vs baseline: 1.0528x; 1.0032x over previous
import functools
import jax
import jax.numpy as jnp
from jax import lax
from jax.experimental import pallas as pl
from jax.experimental.pallas import tpu as pltpu, tpu_sc as plsc

B, J, D = 1024, 26, 1000
ROW = J * D
NBUF = 2

_info = plsc.get_sparse_core_info()
NC, NS, L = _info.num_cores, _info.num_subcores, _info.num_lanes
NW = NC * NS
RPW = B // NW


def _sc_body(x_hbm, out_hbm, x_v, buf0, buf1, sems):
    wid = lax.axis_index("s") * NC + lax.axis_index("c")
    base = wid * RPW
    bufs = [buf0, buf1]

    pltpu.sync_copy(x_hbm.at[pl.ds(base, RPW)], x_v)

    def _zero(i, _):
        z = jnp.zeros((L,), jnp.float32)
        buf0[pl.ds(i * L, L)] = z
        buf1[pl.ds(i * L, L)] = z
        return 0

    lax.fori_loop(0, ROW // L, _zero, 0)

    ones = jnp.ones((L,), jnp.float32)
    zeros = jnp.zeros((L,), jnp.float32)
    lane = lax.broadcasted_iota(jnp.int32, (L,), 0)
    off0 = lane * D
    off1 = (lane + (J - L)) * D

    def _flat_idx(r):
        v0 = x_v[r, pl.ds(0, L)]
        v1 = x_v[r, pl.ds(J - L, L)]
        return off0 + v0, off1 + v1

    copies = [None] * NBUF
    for r in range(RPW):
        b = r % NBUF
        buf = bufs[b]
        if copies[b] is not None:
            copies[b].wait()
            p0, p1 = _flat_idx(r - NBUF)
            plsc.store_scatter(buf, [p0], zeros)
            plsc.store_scatter(buf, [p1], zeros)
        i0, i1 = _flat_idx(r)
        plsc.store_scatter(buf, [i0], ones)
        plsc.store_scatter(buf, [i1], ones)
        cp = pltpu.make_async_copy(buf, out_hbm.at[base + r], sems.at[b])
        cp.start()
        copies[b] = cp
    for b in range(NBUF):
        copies[b].wait()


def kernel(x):
    mesh = plsc.VectorSubcoreMesh(core_axis_name="c", subcore_axis_name="s")
    f = pl.kernel(
        _sc_body,
        out_type=jax.ShapeDtypeStruct((B, ROW), jnp.float32),
        mesh=mesh,
        scratch_types=[
            pltpu.VMEM((RPW, J), jnp.int32),
            pltpu.VMEM((ROW,), jnp.float32),
            pltpu.VMEM((ROW,), jnp.float32),
            pltpu.SemaphoreType.DMA((NBUF,)),
        ],
        compiler_params=pltpu.CompilerParams(
            needs_layout_passes=False, use_tc_tiling_on_sc=True
        ),
    )
    return f(x)


# speedup vs baseline: 2.2071x; 2.0964x over previous
import jax
import jax.numpy as jnp
from jax import lax
from jax.experimental import pallas as pl
from jax.experimental.pallas import tpu as pltpu, tpu_sc as plsc

B, J, D = 1024, 26, 1000
ROWS = J * D
NBANDS = ROWS // 8
BWORDS = 8 * B
NW = 32
NB_BIG = 102
W_BIG = NBANDS - 101 * NW
CH_B = 7
CHW = CH_B * BWORDS
NCH = 15

_info = plsc.get_sparse_core_info()
NC, NS, L = _info.num_cores, _info.num_subcores, _info.num_lanes


def _sc_body(cft_hbm, out_hbm, cf2_v, buf0, buf1, sems):
    wid = lax.axis_index("s") * NC + lax.axis_index("c")
    nb = jnp.where(wid < W_BIG, NB_BIG, NB_BIG - 1)
    blo = NB_BIG * wid - jnp.maximum(wid - W_BIG, 0)
    bhi = blo + nb

    jlo = (blo * 8) // D
    j2 = jnp.minimum(jlo + 1, J - 1)
    pltpu.sync_copy(cft_hbm.at[pl.ds(jlo, 1)], cf2_v.at[pl.ds(0, 1)])
    pltpu.sync_copy(cft_hbm.at[pl.ds(j2, 1)], cf2_v.at[pl.ds(1, 1)])

    def _zero(i, _):
        z = jnp.zeros((L,), jnp.float32)
        buf0[pl.ds(i * L, L)] = z
        buf1[pl.ds(i * L, L)] = z
        return 0

    lax.fori_loop(0, CHW // L, _zero, 0)

    lane = lax.broadcasted_iota(jnp.int32, (L,), 0)
    ones = jnp.ones((L,), jnp.float32)
    zeros = jnp.zeros((L,), jnp.float32)

    def _chunk_lo(m):
        return jnp.minimum(blo + m * CH_B, bhi - CH_B)

    def _scan(buf, m, val):
        rlo = _chunk_lo(m) * 8
        rhi = rlo + CH_B * 8

        def _body(k, _):
            c = k * L + lane
            coff = ((c >> 7) << 10) + (c & 127)
            for jj in (0, 1):
                v = cf2_v[jj, pl.ds(k * L, L)]
                msk = (v >= rlo) & (v < rhi)
                dr = v - rlo
                phys = ((dr >> 3) << 13) + ((dr & 7) << 7) + coff
                plsc.store_scatter(buf, [phys], val, mask=msk)
            return 0

        lax.fori_loop(0, B // L, _body, 0)

    bufs = (buf0, buf1)
    for mg in range(0, NCH + 1, 2):
        for b in range(2):
            m = mg + b
            if m >= NCH:
                continue
            buf = bufs[b]
            dst = out_hbm.at[pl.ds(_chunk_lo(m) * BWORDS, CHW)]
            if m >= 2:
                pltpu.make_async_copy(buf, dst, sems.at[b]).wait()
                _scan(buf, m - 2, zeros)
            _scan(buf, m, ones)
            pltpu.make_async_copy(buf, dst, sems.at[b]).start()
    for b in range(2):
        m = NCH - 2 + ((NCH + b) % 2)
        dst = out_hbm.at[pl.ds(_chunk_lo(m) * BWORDS, CHW)]
        pltpu.make_async_copy(bufs[b], dst, sems.at[b]).wait()


def kernel(x):
    cft = x.T + jnp.arange(J, dtype=x.dtype)[:, None] * D
    mesh = plsc.VectorSubcoreMesh(core_axis_name="c", subcore_axis_name="s")
    f = pl.kernel(
        _sc_body,
        out_type=jax.ShapeDtypeStruct((ROWS * B,), jnp.float32),
        mesh=mesh,
        scratch_types=[
            pltpu.VMEM((2, B), jnp.int32),
            pltpu.VMEM((CHW,), jnp.float32),
            pltpu.VMEM((CHW,), jnp.float32),
            pltpu.SemaphoreType.DMA((2,)),
        ],
        compiler_params=pltpu.CompilerParams(needs_layout_passes=False),
    )
    o = f(cft)
    return o.reshape(NBANDS, 8, 8, 128).transpose(0, 2, 1, 3).reshape(ROWS, B).T
